# baseline (speedup 1.0000x reference)
import functools

import jax
import jax.numpy as jnp
from jax import lax
from jax.experimental import pallas as pl
from jax.experimental.pallas import tpu as pltpu
from jax.experimental.pallas import tpu_sc as plsc

TOTAL_TOKENS = 16384
EMB = 512
_NC = 2
_NS = 16
_NW = _NC * _NS
_BPW = TOTAL_TOKENS // _NW
_CH = 64
_NCHUNK = _BPW // _CH


def _make_gather():
    mesh = plsc.VectorSubcoreMesh(core_axis_name="c", subcore_axis_name="s")

    @functools.partial(
        pl.kernel,
        mesh=mesh,
        out_type=jax.ShapeDtypeStruct((TOTAL_TOKENS, EMB), jnp.float32),
        scratch_types=[
            pltpu.VMEM((_NCHUNK, _CH), jnp.int32),
            pltpu.VMEM((_CH, EMB), jnp.float32),
            pltpu.VMEM((_CH, EMB), jnp.float32),
            pltpu.SemaphoreType.DMA,
            pltpu.SemaphoreType.DMA,
            pltpu.SemaphoreType.DMA,
            pltpu.SemaphoreType.DMA,
        ],
    )
    def gather_k(idx_hbm, table_hbm, out_hbm, idx_v, rows0, rows1,
                 gsem0, gsem1, wsem0, wsem1):
        wid = lax.axis_index("s") * _NC + lax.axis_index("c")
        pltpu.sync_copy(idx_hbm.at[wid], idx_v)
        base = wid * _BPW
        bufs = (rows0, rows1)
        gsems = (gsem0, gsem1)
        wsems = (wsem0, wsem1)
        pending_w = [None, None]
        g = pltpu.async_copy(table_hbm.at[idx_v.at[0]], bufs[0], gsems[0])
        for c in range(_NCHUNK):
            b = c % 2
            g.wait()
            w = pltpu.async_copy(
                bufs[b], out_hbm.at[pl.ds(base + c * _CH, _CH)], wsems[b])
            if c + 1 < _NCHUNK:
                nb = 1 - b
                if pending_w[nb] is not None:
                    pending_w[nb].wait()
                g = pltpu.async_copy(
                    table_hbm.at[idx_v.at[c + 1]], bufs[nb], gsems[nb])
            pending_w[b] = w
        pending_w[0].wait()
        pending_w[1].wait()

    return gather_k


_gather = _make_gather()


def kernel(indices, cu_seqlens, table):
    del cu_seqlens
    idx = indices.astype(jnp.int32).reshape(_NW, _NCHUNK, _CH)
    return _gather(idx, table)


# speedup vs baseline: 1.0236x; 1.0236x over previous
import functools

import jax
import jax.numpy as jnp
from jax import lax
from jax.experimental import pallas as pl
from jax.experimental.pallas import tpu as pltpu
from jax.experimental.pallas import tpu_sc as plsc

TOTAL_TOKENS = 16384
EMB = 512
_NC = 2
_NS = 16
_NW = _NC * _NS
_BPW = TOTAL_TOKENS // _NW
_CH = 128
_NCHUNK = _BPW // _CH


def _make_gather():
    mesh = plsc.VectorSubcoreMesh(core_axis_name="c", subcore_axis_name="s")

    @functools.partial(
        pl.kernel,
        mesh=mesh,
        out_type=jax.ShapeDtypeStruct((TOTAL_TOKENS, EMB), jnp.float32),
        scratch_types=[
            pltpu.VMEM((_BPW,), jnp.int32),
            pltpu.VMEM((_CH, EMB), jnp.float32),
            pltpu.SemaphoreType.DMA,
        ],
    )
    def gather_k(idx_hbm, table_hbm, out_hbm, idx_v, rows_v, sem):
        wid = lax.axis_index("s") * _NC + lax.axis_index("c")
        base = wid * _BPW
        pltpu.sync_copy(idx_hbm.at[pl.ds(base, _BPW)], idx_v)
        for c in range(_NCHUNK):
            pltpu.async_copy(
                table_hbm.at[idx_v.at[pl.ds(c * _CH, _CH)]], rows_v, sem
            ).wait()
            pltpu.sync_copy(rows_v, out_hbm.at[pl.ds(base + c * _CH, _CH)])

    return gather_k


_gather = _make_gather()


def kernel(indices, cu_seqlens, table):
    del cu_seqlens
    return _gather(indices.astype(jnp.int32), table)


# speedup vs baseline: 1.0533x; 1.0290x over previous
import functools

import jax
import jax.numpy as jnp
from jax import lax
from jax.experimental import pallas as pl
from jax.experimental.pallas import tpu as pltpu
from jax.experimental.pallas import tpu_sc as plsc

TOTAL_TOKENS = 16384
EMB = 512
_NC = 2
_NS = 16
_NW = _NC * _NS
_BPW = TOTAL_TOKENS // _NW
_CH = 32
_NCHUNK = _BPW // _CH
_NBUF = 4
_INFLIGHT = 3


def _make_gather():
    mesh = plsc.VectorSubcoreMesh(core_axis_name="c", subcore_axis_name="s")

    @functools.partial(
        pl.kernel,
        mesh=mesh,
        out_type=jax.ShapeDtypeStruct((TOTAL_TOKENS, EMB), jnp.float32),
        scratch_types=[
            pltpu.VMEM((_BPW,), jnp.int32),
        ] + [pltpu.VMEM((_CH, EMB), jnp.float32)] * _NBUF
          + [pltpu.SemaphoreType.DMA] * (2 * _NBUF),
    )
    def gather_k(idx_hbm, table_hbm, out_hbm, idx_v, *bufs_and_sems):
        bufs = bufs_and_sems[:_NBUF]
        gsems = bufs_and_sems[_NBUF:2 * _NBUF]
        wsems = bufs_and_sems[2 * _NBUF:]
        wid = lax.axis_index("s") * _NC + lax.axis_index("c")
        base = wid * _BPW
        pltpu.sync_copy(idx_hbm.at[pl.ds(base, _BPW)], idx_v)

        def start_gather(c):
            slot = c % _NBUF
            return pltpu.async_copy(
                table_hbm.at[idx_v.at[pl.ds(c * _CH, _CH)]],
                bufs[slot], gsems[slot])

        def start_write(c):
            slot = c % _NBUF
            return pltpu.async_copy(
                bufs[slot], out_hbm.at[pl.ds(base + c * _CH, _CH)],
                wsems[slot])

        pend_g = {c: start_gather(c) for c in range(_INFLIGHT)}
        pend_w = {}
        for c in range(_NCHUNK):
            slot = c % _NBUF
            pend_g.pop(c).wait()
            w = start_write(c)
            nc = c + _INFLIGHT
            if nc < _NCHUNK:
                nslot = nc % _NBUF
                if nslot in pend_w:
                    pend_w.pop(nslot).wait()
                pend_g[nc] = start_gather(nc)
            pend_w[slot] = w
        for w in pend_w.values():
            w.wait()

    return gather_k


_gather = _make_gather()


def kernel(indices, cu_seqlens, table):
    del cu_seqlens
    return _gather(indices.astype(jnp.int32), table)


# speedup vs baseline: 1.0818x; 1.0270x over previous
import functools

import jax
import jax.numpy as jnp
from jax import lax
from jax.experimental import pallas as pl
from jax.experimental.pallas import tpu as pltpu
from jax.experimental.pallas import tpu_sc as plsc

TOTAL_TOKENS = 16384
EMB = 512
_NC = 2
_NS = 16
_NW = _NC * _NS
_BPW = TOTAL_TOKENS // _NW
_CH = 32
_NCHUNK = _BPW // _CH
_NBUF = 6
_INFLIGHT = 5


def _make_gather():
    mesh = plsc.VectorSubcoreMesh(core_axis_name="c", subcore_axis_name="s")

    @functools.partial(
        pl.kernel,
        mesh=mesh,
        out_type=jax.ShapeDtypeStruct((TOTAL_TOKENS, EMB), jnp.float32),
        scratch_types=[
            pltpu.VMEM((_BPW,), jnp.int32),
        ] + [pltpu.VMEM((_CH, EMB), jnp.float32)] * _NBUF
          + [pltpu.SemaphoreType.DMA] * (2 * _NBUF),
    )
    def gather_k(idx_hbm, table_hbm, out_hbm, idx_v, *bufs_and_sems):
        bufs = bufs_and_sems[:_NBUF]
        gsems = bufs_and_sems[_NBUF:2 * _NBUF]
        wsems = bufs_and_sems[2 * _NBUF:]
        wid = lax.axis_index("s") * _NC + lax.axis_index("c")
        base = wid * _BPW
        pltpu.sync_copy(idx_hbm.at[pl.ds(base, _BPW)], idx_v)

        def start_gather(c):
            slot = c % _NBUF
            return pltpu.async_copy(
                table_hbm.at[idx_v.at[pl.ds(c * _CH, _CH)]],
                bufs[slot], gsems[slot])

        def start_write(c):
            slot = c % _NBUF
            return pltpu.async_copy(
                bufs[slot], out_hbm.at[pl.ds(base + c * _CH, _CH)],
                wsems[slot])

        pend_g = {c: start_gather(c) for c in range(_INFLIGHT)}
        pend_w = {}
        for c in range(_NCHUNK):
            slot = c % _NBUF
            pend_g.pop(c).wait()
            w = start_write(c)
            nc = c + _INFLIGHT
            if nc < _NCHUNK:
                nslot = nc % _NBUF
                if nslot in pend_w:
                    pend_w.pop(nslot).wait()
                pend_g[nc] = start_gather(nc)
            pend_w[slot] = w
        for w in pend_w.values():
            w.wait()

    return gather_k


_gather = _make_gather()


def kernel(indices, cu_seqlens, table):
    del cu_seqlens
    return _gather(indices.astype(jnp.int32), table)
